# baseline (speedup 1.0000x reference)
import functools

import jax
import jax.numpy as jnp
from jax import lax
from jax.experimental import pallas as pl
from jax.experimental.pallas import tpu as pltpu
from jax.experimental.pallas import tpu_sc as plsc

_NC = 2
_NS = 16
_NW = _NC * _NS

_B = 1024
_T = 20
_V = 35
_E = 64
_ROWS = _T * _V
_RPAD = 768
_CHUNK = 128
_NCHUNK = _RPAD // _CHUNK


def _sc_gather_fn():
    spw = _B // _NW

    mesh = plsc.VectorSubcoreMesh(
        core_axis_name="c", subcore_axis_name="s",
        num_cores=_NC, num_subcores=_NS)

    @functools.partial(
        pl.kernel,
        out_type=jax.ShapeDtypeStruct((_B * _ROWS, 128), jnp.float32),
        mesh=mesh,
        scratch_types=[
            pltpu.VMEM((_NCHUNK, _CHUNK), jnp.int32),
            pltpu.VMEM((_RPAD, 128), jnp.float32),
            pltpu.SemaphoreType.DMA,
        ],
        compiler_params=pltpu.CompilerParams(use_tc_tiling_on_sc=False),
    )
    def sc_gather(idx_hbm, table_hbm, out_hbm, idx_v, y_v, gsem):
        wid = lax.axis_index("s") * _NC + lax.axis_index("c")
        base = wid * spw

        def body(i, carry):
            bb = base + i
            pltpu.sync_copy(idx_hbm.at[pl.ds(bb * _NCHUNK, _NCHUNK)], idx_v)
            cps = []
            for ch in range(_NCHUNK):
                cps.append(pltpu.async_copy(
                    table_hbm.at[idx_v.at[ch]],
                    y_v.at[pl.ds(ch * _CHUNK, _CHUNK)], gsem))
            for cp in cps:
                cp.wait()
            pltpu.sync_copy(y_v.at[pl.ds(0, _ROWS)],
                            out_hbm.at[pl.ds(bb * _ROWS, _ROWS)])
            return carry

        lax.fori_loop(0, spw, body, 0)

    return sc_gather


_sc_gather_cache = []


def _sc_gather(idx, table):
    if not _sc_gather_cache:
        _sc_gather_cache.append(_sc_gather_fn())
    return _sc_gather_cache[0](idx, table)


def _norm_body(g_ref, pe_ref, o_ref):
    g = g_ref[...].reshape(-1, _T, _V, 128)
    y = jnp.concatenate(
        [g[:, :, 0:_V - 1, 0:_E], g[:, :, _V - 1:_V, _E:2 * _E]], axis=2)
    y = y + pe_ref[...][None, :, None, :]
    m = jnp.mean(y, axis=(1, 2, 3), keepdims=True)
    d = y - m
    v = jnp.mean(d * d, axis=(1, 2, 3), keepdims=True)
    o_ref[...] = d * lax.rsqrt(v + 1e-5)


def _norm_fn():
    bb = 8
    return pl.pallas_call(
        _norm_body,
        grid=(_B // bb,),
        in_specs=[
            pl.BlockSpec((bb * _ROWS, 128), lambda i: (i, 0)),
            pl.BlockSpec((_T, _E), lambda i: (0, 0)),
        ],
        out_specs=pl.BlockSpec((bb, _T, _V, _E), lambda i: (i, 0, 0, 0)),
        out_shape=jax.ShapeDtypeStruct((_B, _T, _V, _E), jnp.float32),
    )


_norm = _norm_fn()


def kernel(diag_seq, proc_seq, drug_seq, delta_t, service, admtype, insur,
           marit, seq_length, token_table, pe_dt, pe_pos):
    b, t = delta_t.shape
    vocab, e = token_table.shape

    dt = delta_t / 15.0
    len_mask = jnp.arange(t)[None, :] < seq_length[:, None]
    dt = jnp.cumsum(jnp.round(dt), axis=1) * len_mask.astype(dt.dtype)
    dt_idx = jnp.clip(dt.astype(jnp.int32), 0, pe_dt.shape[0] - 1)

    tok34 = jnp.concatenate(
        [diag_seq, proc_seq, drug_seq, service, admtype,
         insur[..., None], marit[..., None]], axis=2)
    idx35 = jnp.concatenate([tok34, dt_idx[..., None]], axis=2)
    idx = jnp.pad(idx35.reshape(b, _ROWS), ((0, 0), (0, _RPAD - _ROWS)))
    idx = idx.reshape(b * _NCHUNK, _CHUNK)

    table128 = jnp.concatenate([token_table, pe_dt], axis=1)

    g = _sc_gather(idx, table128)

    return _norm(g, pe_pos[:t])
